# baseline (speedup 1.0000x reference)
import functools

import jax
import jax.numpy as jnp
from jax import lax
from jax.experimental import pallas as pl
from jax.experimental.pallas import tpu as pltpu
from jax.experimental.pallas import tpu_sc as plsc

_NUM_EMB = 197
_DIM = 768
_SPLIT_ROW = 96
_MAX_ROWS = _NUM_EMB - _SPLIT_ROW
_NUM_SUBCORES = 16


def _broadcast_table_sc(W, batch):
    b_per_tile = batch // _NUM_SUBCORES
    mesh = plsc.VectorSubcoreMesh(core_axis_name="c", subcore_axis_name="s")

    @functools.partial(
        pl.kernel,
        out_type=jax.ShapeDtypeStruct((batch, _NUM_EMB, _DIM), W.dtype),
        mesh=mesh,
        scratch_types=[
            pltpu.VMEM((_MAX_ROWS, _DIM), W.dtype),
            pltpu.SemaphoreType.DMA,
        ],
    )
    def k(w_hbm, out_hbm, w_tile, sem):
        core = lax.axis_index("c")
        sub = lax.axis_index("s")
        base = sub * b_per_tile

        def do_half(r0, nrows):
            wt = w_tile.at[pl.ds(0, nrows), :]
            pltpu.sync_copy(w_hbm.at[pl.ds(r0, nrows), :], wt)

            @pl.loop(0, b_per_tile)
            def _(i):
                pltpu.async_copy(
                    wt, out_hbm.at[base + i, pl.ds(r0, nrows), :], sem
                )

            @pl.loop(0, b_per_tile)
            def _(i):
                pltpu.make_async_copy(
                    wt, out_hbm.at[base + i, pl.ds(r0, nrows), :], sem
                ).wait()

        @pl.when(core == 0)
        def _():
            do_half(0, _SPLIT_ROW)

        @pl.when(core == 1)
        def _():
            do_half(_SPLIT_ROW, _MAX_ROWS)

    return k(W)


def kernel(x, W):
    return _broadcast_table_sc(W, x.shape[0])
